# baseline (speedup 1.0000x reference)
import jax
import jax.numpy as jnp
from jax import lax
from jax.experimental import pallas as pl
from jax.experimental.pallas import tpu as pltpu
from jax.experimental.pallas import tpu_sc as plsc

BATCH = 16384
D = 64
NC = 2
NS = 16
NW = NC * NS
BPW = BATCH // NW
VG = BPW // 16


def _gather_one(idx_v, tab, out, base, stage, buf, sem0, sem1):
    cap = tab.shape[1] - 1
    sems = (sem0, sem1)
    row16 = lax.broadcasted_iota(jnp.int32, (16,), 0)

    def loadv(g):
        vi = pl.multiple_of(g * 16, 16)
        return jnp.minimum(idx_v[pl.ds(vi, 16)], cap)

    def fire(v16, s):
        for k in range(4):
            c = pl.multiple_of((v16[s * 4 + k] >> 7) * 128, 128)
            for tr in range(8):
                rows = pl.ds(tr * 8, 8)
                pltpu.async_copy(tab.at[rows, pl.ds(c, 128)],
                                 buf.at[s % 2, k, rows], sems[s % 2])

    def drain(s):
        for k in range(4):
            pltpu.make_async_copy(tab.at[:, pl.ds(0, 128)],
                                  buf.at[s % 2, k], sems[s % 2]).wait()

    def extract(g, v16, s):
        for k in range(4):
            e_par = k % 2
            row = g * 8 + s * 2 + k // 2
            ce = jnp.full((16,), v16[s * 4 + k] & 127, jnp.int32)
            for h in range(4):
                vals = plsc.load_gather(buf.at[s % 2, k],
                                        [row16 + h * 16, ce])
                stage[row, pl.ds(e_par * D + h * 16, 16)] = vals

    def body(g, _):
        v16 = loadv(g)
        fire(v16, 0)

        @pl.when(g > 0)
        def _():
            v16p = loadv(g - 1)
            drain(3)
            extract(g - 1, v16p, 3)

        fire(v16, 1)
        drain(0)
        extract(g, v16, 0)
        fire(v16, 2)
        drain(1)
        extract(g, v16, 1)
        fire(v16, 3)
        drain(2)
        extract(g, v16, 2)
        return 0

    lax.fori_loop(0, VG, body, 0)
    v16l = loadv(VG - 1)
    drain(3)
    extract(VG - 1, v16l, 3)
    pltpu.sync_copy(stage, out.at[pl.ds(pl.multiple_of(base // 2, 8),
                                        BPW // 2)])


def _gather_body(p_idx_hbm, t_idx_hbm, p_tab, t_tab, p_out, t_out,
                 pidx_v, tidx_v, buf, stage, sem0, sem1):
    wid = lax.axis_index("s") * NC + lax.axis_index("c")
    base = wid * BPW
    pltpu.sync_copy(p_idx_hbm.at[pl.ds(base, BPW)], pidx_v)
    pltpu.sync_copy(t_idx_hbm.at[pl.ds(base, BPW)], tidx_v)
    _gather_one(pidx_v, p_tab, p_out, base, stage, buf, sem0, sem1)
    _gather_one(tidx_v, t_tab, t_out, base, stage, buf, sem0, sem1)


def _sc_gather(p_idx, t_idx, p_tabT, t_tabT):
    mesh = plsc.VectorSubcoreMesh(core_axis_name="c", subcore_axis_name="s")
    k = pl.kernel(
        _gather_body,
        out_type=[jax.ShapeDtypeStruct((BATCH // 2, 2 * D), jnp.float32),
                  jax.ShapeDtypeStruct((BATCH // 2, 2 * D), jnp.float32)],
        mesh=mesh,
        scratch_types=[
            pltpu.VMEM((BPW,), jnp.int32),
            pltpu.VMEM((BPW,), jnp.int32),
            pltpu.VMEM((2, 4, D, 128), jnp.float32),
            pltpu.VMEM((BPW // 2, 2 * D), jnp.float32),
            pltpu.SemaphoreType.DMA,
            pltpu.SemaphoreType.DMA,
        ],
        compiler_params=pltpu.CompilerParams(needs_layout_passes=False),
    )
    return k(p_idx, t_idx, p_tabT, t_tabT)


def _mlp_body(p_ref, t_ref, wp_ref, wt_ref, bp_ref, bt_ref, o_ref):
    xl, xr = p_ref[:, :D], p_ref[:, D:]
    yl, yr = t_ref[:, :D], t_ref[:, D:]
    dn = (((1,), (1,)), ((), ()))
    pvl = jnp.maximum(lax.dot_general(xl, wp_ref[...], dn,
                      preferred_element_type=jnp.float32) + bp_ref[...], 0.)
    pvr = jnp.maximum(lax.dot_general(xr, wp_ref[...], dn,
                      preferred_element_type=jnp.float32) + bp_ref[...], 0.)
    tvl = jnp.maximum(lax.dot_general(yl, wt_ref[...], dn,
                      preferred_element_type=jnp.float32) + bt_ref[...], 0.)
    tvr = jnp.maximum(lax.dot_general(yr, wt_ref[...], dn,
                      preferred_element_type=jnp.float32) + bt_ref[...], 0.)
    even = jnp.sum(pvl * tvl, axis=1, keepdims=True)
    odd = jnp.sum(pvr * tvr, axis=1, keepdims=True)
    o_ref[...] = jnp.concatenate([even, odd], axis=1)


def _tc_mlp_dot(p_emb2, t_emb2, Wp, bp, Wt, bt):
    nrow = BATCH // 2
    nblk = 16
    blk = nrow // nblk
    out = pl.pallas_call(
        _mlp_body,
        grid=(nblk,),
        in_specs=[
            pl.BlockSpec((blk, 128), lambda i: (i, 0)),
            pl.BlockSpec((blk, 128), lambda i: (i, 0)),
            pl.BlockSpec((D, D), lambda i: (0, 0)),
            pl.BlockSpec((D, D), lambda i: (0, 0)),
            pl.BlockSpec((1, D), lambda i: (0, 0)),
            pl.BlockSpec((1, D), lambda i: (0, 0)),
        ],
        out_specs=pl.BlockSpec((blk, 2), lambda i: (i, 0)),
        out_shape=jax.ShapeDtypeStruct((nrow, 2), jnp.float32),
    )(p_emb2, t_emb2, Wp, Wt, bp.reshape(1, D), bt.reshape(1, D))
    return out.reshape(BATCH)


def kernel(p_idx, t_idx, play_table, track_table, Wp, bp, Wt, bt):
    p_emb2, t_emb2 = _sc_gather(p_idx, t_idx, play_table.T, track_table.T)
    return _tc_mlp_dot(p_emb2, t_emb2, Wp, bp, Wt, bt)


# speedup vs baseline: 1.0055x; 1.0055x over previous
import jax
import jax.numpy as jnp
from jax import lax
from jax.experimental import pallas as pl
from jax.experimental.pallas import tpu as pltpu
from jax.experimental.pallas import tpu_sc as plsc

BATCH = 16384
D = 64
NC = 2
NS = 16
NW = NC * NS
BPW = BATCH // NW
VG = BPW // 16


def _gather_one(idx_v, tab, out, base, stage, buf, sem0, sem1):
    cap = tab.shape[1] - 1
    sems = (sem0, sem1)
    row16 = lax.broadcasted_iota(jnp.int32, (16,), 0)

    def loadv(g):
        vi = pl.multiple_of(g * 16, 16)
        return jnp.minimum(idx_v[pl.ds(vi, 16)], cap)

    def fire(v16, s):
        for k in range(4):
            c = pl.multiple_of((v16[s * 4 + k] >> 7) * 128, 128)
            pltpu.async_copy(tab.at[:, pl.ds(c, 128)],
                             buf.at[s % 2, k], sems[s % 2])

    def drain(s):
        for k in range(4):
            pltpu.make_async_copy(tab.at[:, pl.ds(0, 128)],
                                  buf.at[s % 2, k], sems[s % 2]).wait()

    def extract(g, v16, s):
        for k in range(4):
            e_par = k % 2
            row = g * 8 + s * 2 + k // 2
            ce = jnp.full((16,), v16[s * 4 + k] & 127, jnp.int32)
            for h in range(4):
                vals = plsc.load_gather(buf.at[s % 2, k],
                                        [row16 + h * 16, ce])
                stage[row, pl.ds(e_par * D + h * 16, 16)] = vals

    def body(g, _):
        v16 = loadv(g)
        fire(v16, 0)

        @pl.when(g > 0)
        def _():
            v16p = loadv(g - 1)
            drain(3)
            extract(g - 1, v16p, 3)

        fire(v16, 1)
        drain(0)
        extract(g, v16, 0)
        fire(v16, 2)
        drain(1)
        extract(g, v16, 1)
        fire(v16, 3)
        drain(2)
        extract(g, v16, 2)
        return 0

    lax.fori_loop(0, VG, body, 0)
    v16l = loadv(VG - 1)
    drain(3)
    extract(VG - 1, v16l, 3)
    pltpu.sync_copy(stage, out.at[pl.ds(pl.multiple_of(base // 2, 8),
                                        BPW // 2)])


def _gather_body(p_idx_hbm, t_idx_hbm, p_tab, t_tab, p_out, t_out,
                 pidx_v, tidx_v, buf, stage, sem0, sem1):
    wid = lax.axis_index("s") * NC + lax.axis_index("c")
    base = wid * BPW
    pltpu.sync_copy(p_idx_hbm.at[pl.ds(base, BPW)], pidx_v)
    pltpu.sync_copy(t_idx_hbm.at[pl.ds(base, BPW)], tidx_v)
    _gather_one(pidx_v, p_tab, p_out, base, stage, buf, sem0, sem1)
    _gather_one(tidx_v, t_tab, t_out, base, stage, buf, sem0, sem1)


def _sc_gather(p_idx, t_idx, p_tabT, t_tabT):
    mesh = plsc.VectorSubcoreMesh(core_axis_name="c", subcore_axis_name="s")
    k = pl.kernel(
        _gather_body,
        out_type=[jax.ShapeDtypeStruct((BATCH // 2, 2 * D), jnp.float32),
                  jax.ShapeDtypeStruct((BATCH // 2, 2 * D), jnp.float32)],
        mesh=mesh,
        scratch_types=[
            pltpu.VMEM((BPW,), jnp.int32),
            pltpu.VMEM((BPW,), jnp.int32),
            pltpu.VMEM((2, 4, D, 128), jnp.float32),
            pltpu.VMEM((BPW // 2, 2 * D), jnp.float32),
            pltpu.SemaphoreType.DMA,
            pltpu.SemaphoreType.DMA,
        ],
        compiler_params=pltpu.CompilerParams(needs_layout_passes=False),
    )
    return k(p_idx, t_idx, p_tabT, t_tabT)


def _mlp_body(p_ref, t_ref, wp_ref, wt_ref, bp_ref, bt_ref, o_ref):
    xl, xr = p_ref[:, :D], p_ref[:, D:]
    yl, yr = t_ref[:, :D], t_ref[:, D:]
    dn = (((1,), (1,)), ((), ()))
    pvl = jnp.maximum(lax.dot_general(xl, wp_ref[...], dn,
                      preferred_element_type=jnp.float32) + bp_ref[...], 0.)
    pvr = jnp.maximum(lax.dot_general(xr, wp_ref[...], dn,
                      preferred_element_type=jnp.float32) + bp_ref[...], 0.)
    tvl = jnp.maximum(lax.dot_general(yl, wt_ref[...], dn,
                      preferred_element_type=jnp.float32) + bt_ref[...], 0.)
    tvr = jnp.maximum(lax.dot_general(yr, wt_ref[...], dn,
                      preferred_element_type=jnp.float32) + bt_ref[...], 0.)
    even = jnp.sum(pvl * tvl, axis=1, keepdims=True)
    odd = jnp.sum(pvr * tvr, axis=1, keepdims=True)
    o_ref[...] = jnp.concatenate([even, odd], axis=1)


def _tc_mlp_dot(p_emb2, t_emb2, Wp, bp, Wt, bt):
    nrow = BATCH // 2
    nblk = 16
    blk = nrow // nblk
    out = pl.pallas_call(
        _mlp_body,
        grid=(nblk,),
        in_specs=[
            pl.BlockSpec((blk, 128), lambda i: (i, 0)),
            pl.BlockSpec((blk, 128), lambda i: (i, 0)),
            pl.BlockSpec((D, D), lambda i: (0, 0)),
            pl.BlockSpec((D, D), lambda i: (0, 0)),
            pl.BlockSpec((1, D), lambda i: (0, 0)),
            pl.BlockSpec((1, D), lambda i: (0, 0)),
        ],
        out_specs=pl.BlockSpec((blk, 2), lambda i: (i, 0)),
        out_shape=jax.ShapeDtypeStruct((nrow, 2), jnp.float32),
    )(p_emb2, t_emb2, Wp, Wt, bp.reshape(1, D), bt.reshape(1, D))
    return out.reshape(BATCH)


def kernel(p_idx, t_idx, play_table, track_table, Wp, bp, Wt, bt):
    p_emb2, t_emb2 = _sc_gather(p_idx, t_idx, play_table.T, track_table.T)
    return _tc_mlp_dot(p_emb2, t_emb2, Wp, bp, Wt, bt)
